# baseline (speedup 1.0000x reference)
import functools

import jax
import jax.numpy as jnp
from jax import lax
from jax.experimental import pallas as pl
from jax.experimental.pallas import tpu as pltpu
from jax.experimental.pallas import tpu_sc as plsc

NC = 2
NS = 16
NW = NC * NS

CH = 128
NBUF = 8


@functools.lru_cache(maxsize=None)
def _make_lookup(n_chunks_total, vocab, dim):
    n_chunks = n_chunks_total // NW
    assert n_chunks % NBUF == 0
    n_rows = n_chunks_total * CH

    mesh = plsc.VectorSubcoreMesh(core_axis_name="c", subcore_axis_name="s")

    @functools.partial(
        pl.kernel,
        mesh=mesh,
        compiler_params=pltpu.CompilerParams(use_tc_tiling_on_sc=False),
        out_type=jax.ShapeDtypeStruct((n_rows, dim), jnp.float32),
        scratch_types=[
            pltpu.VMEM((n_chunks, CH), jnp.int32),
            pltpu.VMEM((NBUF, CH, dim), jnp.float32),
            pltpu.SemaphoreType.DMA((NBUF,)),
            pltpu.SemaphoreType.DMA((NBUF,)),
        ],
    )
    def lookup(x_hbm, w_hbm, out_hbm, idx_v, rows_v, gsem, ssem):
        wid = lax.axis_index("s") * NC + lax.axis_index("c")
        chunk0 = wid * n_chunks
        out0 = chunk0 * CH

        pltpu.sync_copy(x_hbm.at[pl.ds(chunk0, n_chunks)], idx_v)

        def gather(g, b):
            return pltpu.make_async_copy(
                w_hbm.at[idx_v.at[g]], rows_v.at[b], gsem.at[b])

        def scatter(g, b):
            return pltpu.make_async_copy(
                rows_v.at[b],
                out_hbm.at[pl.ds(out0 + g * CH, CH)],
                ssem.at[b])

        for b in range(NBUF):
            gather(b, b).start()

        def group(i, _):
            i0 = i * NBUF
            for b in range(NBUF):
                g = i0 + b
                gather(g, b).wait()
                scatter(g, b).start()
                f = g + NBUF

                @pl.when(f < n_chunks)
                def _():
                    scatter(g, b).wait()
                    gather(f, b).start()
            return 0

        lax.fori_loop(0, n_chunks // NBUF, group, 0)

        for b in range(NBUF):
            scatter(n_chunks - NBUF + b, b).wait()

    return lookup


def kernel(x, weight):
    b0, b1 = x.shape
    n = b0 * b1
    xf = x.reshape(n // CH, CH).astype(jnp.int32)
    out = _make_lookup(n // CH, weight.shape[0], weight.shape[1])(xf, weight)
    return out.reshape(b0, b1, weight.shape[1])
